# baseline (speedup 1.0000x reference)
import functools

import jax
import jax.numpy as jnp
import numpy as np
from jax import lax
from jax.experimental import pallas as pl
from jax.experimental.pallas import tpu as pltpu

_A = 6
_S = 20
_OPP = 2


def _rotl(x, r):
    return (x << np.uint32(r)) | (x >> np.uint32(32 - r))


def _threefry2x32(k1, k2, x0, x1):
    ks0 = np.uint32(k1)
    ks1 = np.uint32(k2)
    ks2 = np.uint32(ks0 ^ ks1 ^ np.uint32(0x1BD11BDA))
    ks = [ks0, ks1, ks2]
    rotations = [(13, 15, 26, 6), (17, 29, 16, 24)]
    x0 = x0 + ks0
    x1 = x1 + ks1
    for i in range(5):
        for r in rotations[i % 2]:
            x0 = x0 + x1
            x1 = _rotl(x1, r)
            x1 = x1 ^ x0
        x0 = x0 + ks[(i + 1) % 3]
        x1 = x1 + ks[(i + 2) % 3] + np.uint32(i + 1)
    return x0, x1


def _fold_in(key, data):
    o0, o1 = _threefry2x32(key[0], key[1],
                           np.atleast_1d(np.uint32(0)),
                           np.atleast_1d(np.uint32(data)))
    return (o0[0], o1[0])


def _gumbel_np(key, n):
    cnt = np.arange(n, dtype=np.uint64)
    hi = (cnt >> np.uint64(32)).astype(np.uint32)
    lo = (cnt & np.uint64(0xFFFFFFFF)).astype(np.uint32)
    o0, o1 = _threefry2x32(key[0], key[1], hi, lo)
    bits = o0 ^ o1
    f = ((bits >> np.uint32(9)) | np.uint32(0x3F800000)).view(np.float32)
    u = f - np.float32(1.0)
    tiny = np.float32(np.finfo(np.float32).tiny)
    u = np.maximum(tiny, u * (np.float32(1.0) - tiny) + tiny)
    with np.errstate(divide="ignore"):
        return -np.log(-np.log(u))


@functools.lru_cache(maxsize=2)
def _gumbel_host_b(B, Bb):
    root = (np.uint32(0), np.uint32(42))
    nb = B // Bb
    out = np.empty((nb, _OPP * _A, _S, Bb), np.float32)
    for j in range(_OPP):
        kj = _fold_in(root, j)
        for i in range(_S):
            ki = _fold_in(kj, i)
            g = _gumbel_np(ki, B * _A).reshape(nb, Bb, _A)
            for a in range(_A):
                out[:, j * _A + a, i, :] = g[:, :, a]
    return out


def _body_b(xb_ref, wcat_ref, bcat_ref, c0_ref, c1_ref, g_ref, out_ref):
    A, S = _A, _S
    xb = xb_ref[...]
    zz = lax.dot_general(wcat_ref[...], xb, (((1,), (1,)), ((), ())),
                         preferred_element_type=jnp.float32)
    zz = zz + bcat_ref[...]
    Bb = xb.shape[0]

    z0 = [zz[a:a + 1, :] for a in range(A)]
    z1 = [zz[8 + a:9 + a, :] for a in range(A)]
    bs = [zz[16 + o:17 + o, :] for o in range(A)]
    m0 = z0[0]
    m1 = z1[0]
    for a in range(1, A):
        m0 = jnp.maximum(m0, z0[a])
        m1 = jnp.maximum(m1, z1[a])
    mm = m0 + m1

    c0 = c0_ref[...]
    c1 = c1_ref[...]

    def pick(zrows, goff):
        v = [zrows[a] + g_ref[0, goff + a, :, :] for a in range(A)]
        m = v[0]
        for a in range(1, A):
            m = jnp.maximum(m, v[a])
        hit = [v[a] == m for a in range(A)]
        u = jnp.broadcast_to(zrows[A - 1], (S, Bb))
        for a in range(A - 2, -1, -1):
            u = jnp.where(hit[a], zrows[a], u)
        return hit, u

    hit0, u0 = pick(z0, 0)
    hit1, u1 = pick(z1, A)
    w = jnp.exp(u0 + u1 - mm)

    s = None
    e = []
    for o in range(A):
        ebs = jnp.exp(bs[o])
        d = jnp.broadcast_to(c0[o, A - 1] * ebs, (S, Bb))
        for a in range(A - 2, -1, -1):
            d = jnp.where(hit0[a], c0[o, a] * ebs, d)
        dd = jnp.broadcast_to(c1[o, A - 1], (S, Bb))
        for a in range(A - 2, -1, -1):
            dd = jnp.where(hit1[a], c1[o, a], dd)
        eo = d * dd
        e.append(eo)
        s = eo if s is None else s + eo
    r = w / s
    wsum = jnp.sum(w, axis=0, keepdims=True)
    outs = []
    for o in range(A):
        outs.append(jnp.sum(r * e[o], axis=0, keepdims=True) / wsum)
    out_ref[...] = jnp.concatenate(outs, axis=0).T


def kernel(x, W_opp0, b_opp0, W_opp1, b_opp1, W, b):
    B, D = x.shape
    A, S = _A, _S

    Bb = 8192
    nb = B // Bb
    g = jnp.asarray(_gumbel_host_b(B, Bb))

    zpadW = jnp.zeros((2, D), x.dtype)
    wcat = jnp.concatenate(
        [W_opp0, zpadW, W_opp1, zpadW, W[:, :D], zpadW], axis=0)
    zpadb = jnp.zeros((2,), x.dtype)
    bcat = jnp.concatenate(
        [b_opp0, zpadb, b_opp1, zpadb, b, zpadb], axis=0)[:, None]
    c0 = jnp.exp(W[:, D:D + A])
    c1 = jnp.exp(W[:, D + A:D + 2 * A])

    out = pl.pallas_call(
        _body_b,
        grid=(nb,),
        in_specs=[
            pl.BlockSpec((Bb, D), lambda i: (i, 0)),
            pl.BlockSpec((24, D), lambda i: (0, 0)),
            pl.BlockSpec((24, 1), lambda i: (0, 0)),
            pl.BlockSpec((A, A), lambda i: (0, 0)),
            pl.BlockSpec((A, A), lambda i: (0, 0)),
            pl.BlockSpec((1, _OPP * A, S, Bb), lambda i: (i, 0, 0, 0)),
        ],
        out_specs=pl.BlockSpec((Bb, A), lambda i: (i, 0)),
        out_shape=jax.ShapeDtypeStruct((B, A), jnp.float32),
        compiler_params=pltpu.CompilerParams(
            dimension_semantics=("parallel",),
        ),
    )(x, wcat, bcat, c0, c1, g)
    return out


# speedup vs baseline: 1.0523x; 1.0523x over previous
import functools

import jax
import jax.numpy as jnp
import numpy as np
from jax import lax
from jax.experimental import pallas as pl
from jax.experimental.pallas import tpu as pltpu

_A = 6
_S = 20
_OPP = 2


def _rotl(x, r):
    return (x << np.uint32(r)) | (x >> np.uint32(32 - r))


def _threefry2x32(k1, k2, x0, x1):
    ks0 = np.uint32(k1)
    ks1 = np.uint32(k2)
    ks2 = np.uint32(ks0 ^ ks1 ^ np.uint32(0x1BD11BDA))
    ks = [ks0, ks1, ks2]
    rotations = [(13, 15, 26, 6), (17, 29, 16, 24)]
    x0 = x0 + ks0
    x1 = x1 + ks1
    for i in range(5):
        for r in rotations[i % 2]:
            x0 = x0 + x1
            x1 = _rotl(x1, r)
            x1 = x1 ^ x0
        x0 = x0 + ks[(i + 1) % 3]
        x1 = x1 + ks[(i + 2) % 3] + np.uint32(i + 1)
    return x0, x1


def _fold_in(key, data):
    o0, o1 = _threefry2x32(key[0], key[1],
                           np.atleast_1d(np.uint32(0)),
                           np.atleast_1d(np.uint32(data)))
    return (o0[0], o1[0])


def _gumbel_np(key, n):
    cnt = np.arange(n, dtype=np.uint64)
    hi = (cnt >> np.uint64(32)).astype(np.uint32)
    lo = (cnt & np.uint64(0xFFFFFFFF)).astype(np.uint32)
    o0, o1 = _threefry2x32(key[0], key[1], hi, lo)
    bits = o0 ^ o1
    f = ((bits >> np.uint32(9)) | np.uint32(0x3F800000)).view(np.float32)
    u = f - np.float32(1.0)
    tiny = np.float32(np.finfo(np.float32).tiny)
    u = np.maximum(tiny, u * (np.float32(1.0) - tiny) + tiny)
    with np.errstate(divide="ignore"):
        return -np.log(-np.log(u))


@functools.lru_cache(maxsize=2)
def _gumbel_host_b(B, Bb):
    root = (np.uint32(0), np.uint32(42))
    nb = B // Bb
    out = np.empty((nb, _OPP * _A, _S, Bb), np.float32)
    for j in range(_OPP):
        kj = _fold_in(root, j)
        for i in range(_S):
            ki = _fold_in(kj, i)
            g = _gumbel_np(ki, B * _A).reshape(nb, Bb, _A)
            for a in range(_A):
                out[:, j * _A + a, i, :] = g[:, :, a]
    return out


def _body_b(xb_ref, wcat_ref, bcat_ref, c0_ref, c1_ref, g_ref, out_ref):
    A, S = _A, _S
    xb = xb_ref[...]
    zz = lax.dot_general(wcat_ref[...], xb, (((1,), (1,)), ((), ())),
                         preferred_element_type=jnp.float32)
    zz = zz + bcat_ref[...]
    Bb = xb.shape[0]

    z0 = [zz[a:a + 1, :] for a in range(A)]
    z1 = [zz[8 + a:9 + a, :] for a in range(A)]
    bs = [zz[16 + o:17 + o, :] for o in range(A)]
    m0 = z0[0]
    m1 = z1[0]
    for a in range(1, A):
        m0 = jnp.maximum(m0, z0[a])
        m1 = jnp.maximum(m1, z1[a])
    mm = m0 + m1

    c0 = c0_ref[...]
    c1 = c1_ref[...]

    def pick(zrows, goff):
        v = [zrows[a] + g_ref[0, goff + a, :, :] for a in range(A)]
        m = v[0]
        for a in range(1, A):
            m = jnp.maximum(m, v[a])
        hit = [v[a] == m for a in range(A)]
        u = jnp.broadcast_to(zrows[A - 1], (S, Bb))
        for a in range(A - 2, -1, -1):
            u = jnp.where(hit[a], zrows[a], u)
        return hit, u

    hit0, u0 = pick(z0, 0)
    hit1, u1 = pick(z1, A)
    w = jnp.exp(u0 + u1 - mm)

    s = None
    e = []
    for o in range(A):
        ebs = jnp.exp(bs[o])
        d = jnp.broadcast_to(c0[o, A - 1] * ebs, (S, Bb))
        for a in range(A - 2, -1, -1):
            d = jnp.where(hit0[a], c0[o, a] * ebs, d)
        dd = jnp.broadcast_to(c1[o, A - 1], (S, Bb))
        for a in range(A - 2, -1, -1):
            dd = jnp.where(hit1[a], c1[o, a], dd)
        eo = d * dd
        e.append(eo)
        s = eo if s is None else s + eo
    r = w / s
    wsum = jnp.sum(w, axis=0, keepdims=True)
    outs = []
    for o in range(A):
        outs.append(jnp.sum(r * e[o], axis=0, keepdims=True) / wsum)
    out_ref[...] = jnp.concatenate(outs, axis=0).T


def kernel(x, W_opp0, b_opp0, W_opp1, b_opp1, W, b):
    B, D = x.shape
    A, S = _A, _S

    Bb = 4096
    nb = B // Bb
    g = jnp.asarray(_gumbel_host_b(B, Bb))

    zpadW = jnp.zeros((2, D), x.dtype)
    wcat = jnp.concatenate(
        [W_opp0, zpadW, W_opp1, zpadW, W[:, :D], zpadW], axis=0)
    zpadb = jnp.zeros((2,), x.dtype)
    bcat = jnp.concatenate(
        [b_opp0, zpadb, b_opp1, zpadb, b, zpadb], axis=0)[:, None]
    c0 = jnp.exp(W[:, D:D + A])
    c1 = jnp.exp(W[:, D + A:D + 2 * A])

    out = pl.pallas_call(
        _body_b,
        grid=(nb,),
        in_specs=[
            pl.BlockSpec((Bb, D), lambda i: (i, 0)),
            pl.BlockSpec((24, D), lambda i: (0, 0)),
            pl.BlockSpec((24, 1), lambda i: (0, 0)),
            pl.BlockSpec((A, A), lambda i: (0, 0)),
            pl.BlockSpec((A, A), lambda i: (0, 0)),
            pl.BlockSpec((1, _OPP * A, S, Bb), lambda i: (i, 0, 0, 0)),
        ],
        out_specs=pl.BlockSpec((Bb, A), lambda i: (i, 0)),
        out_shape=jax.ShapeDtypeStruct((B, A), jnp.float32),
        compiler_params=pltpu.CompilerParams(
            dimension_semantics=("parallel",),
        ),
    )(x, wcat, bcat, c0, c1, g)
    return out
